# baseline (speedup 1.0000x reference)
import jax
import jax.numpy as jnp
from jax.experimental import pallas as pl
from jax.experimental.pallas import tpu as pltpu

_BP = 512


def _vq_onehot_body(xf_ref, c_ref, out_ref):
    c = c_ref[...]
    d = jax.lax.dot_general(
        xf_ref[...].astype(jnp.bfloat16), c.astype(jnp.bfloat16),
        dimension_numbers=(((1,), (1,)), ((), ())),
        preferred_element_type=jnp.float32,
    )
    ones = jnp.ones((1, c.shape[1]), jnp.float32)
    csq = jax.lax.dot_general(
        ones, c * c,
        dimension_numbers=(((1,), (1,)), ((), ())),
        preferred_element_type=jnp.float32,
        precision=jax.lax.Precision.HIGHEST,
    )
    scores = csq - 2.0 * d
    labels = jnp.argmin(scores, axis=1)
    iota = jax.lax.broadcasted_iota(jnp.int32, scores.shape, 1)
    out_ref[...] = (iota == labels[:, None]).astype(jnp.float32)


def kernel(x, cluster_centers):
    H, W, C = x.shape
    K = cluster_centers.shape[0]
    P = H * W
    xf = x.reshape(P, C)
    grid = P // _BP

    out = pl.pallas_call(
        _vq_onehot_body,
        grid=(grid,),
        in_specs=[
            pl.BlockSpec((_BP, C), lambda i: (i, 0)),
            pl.BlockSpec((K, C), lambda i: (0, 0)),
        ],
        out_specs=pl.BlockSpec((_BP, K), lambda i: (i, 0)),
        out_shape=jax.ShapeDtypeStruct((P, K), jnp.float32),
    )(xf, cluster_centers)
    return out.reshape(H, W, K)


# speedup vs baseline: 1.8824x; 1.8824x over previous
import jax
import jax.numpy as jnp
from jax.experimental import pallas as pl
from jax.experimental.pallas import tpu as pltpu

_BP = 512


def _vq_onehot_body(xf_ref, c_ref, out_ref, cb_ref, csq_ref):
    @pl.when(pl.program_id(0) == 0)
    def _():
        c = c_ref[...]
        cb_ref[...] = c.astype(jnp.bfloat16)
        ones = jnp.ones((1, c.shape[1]), jnp.float32)
        csq_ref[...] = jax.lax.dot_general(
            ones, c * c,
            dimension_numbers=(((1,), (1,)), ((), ())),
            preferred_element_type=jnp.float32,
            precision=jax.lax.Precision.HIGHEST,
        )

    xf = xf_ref[...]
    d = jax.lax.dot_general(
        xf.astype(jnp.bfloat16), cb_ref[...],
        dimension_numbers=(((1,), (1,)), ((), ())),
        preferred_element_type=jnp.float32,
    )
    xsq = jnp.sum(xf * xf, axis=1, keepdims=True)
    dists = (xsq - 2.0 * d) + csq_ref[...]
    minv = jnp.min(dists, axis=1, keepdims=True)
    out_ref[...] = jnp.where(dists == minv, 1.0, 0.0).astype(jnp.float32)


def kernel(x, cluster_centers):
    H, W, C = x.shape
    K = cluster_centers.shape[0]
    P = H * W
    xf = x.reshape(P, C)
    grid = P // _BP

    out = pl.pallas_call(
        _vq_onehot_body,
        grid=(grid,),
        in_specs=[
            pl.BlockSpec((_BP, C), lambda i: (i, 0)),
            pl.BlockSpec((K, C), lambda i: (0, 0)),
        ],
        out_specs=pl.BlockSpec((_BP, K), lambda i: (i, 0)),
        out_shape=jax.ShapeDtypeStruct((P, K), jnp.float32),
        scratch_shapes=[
            pltpu.VMEM((K, C), jnp.bfloat16),
            pltpu.VMEM((1, K), jnp.float32),
        ],
    )(xf, cluster_centers)
    return out.reshape(H, W, K)


# speedup vs baseline: 1.9193x; 1.0196x over previous
import jax
import jax.numpy as jnp
from jax.experimental import pallas as pl
from jax.experimental.pallas import tpu as pltpu

_BP = 512


def _vq_onehot_body(xf_ref, c_ref, out_ref, cb_ref, csqh_ref):
    @pl.when(pl.program_id(0) == 0)
    def _():
        c = c_ref[...]
        cb_ref[...] = c.astype(jnp.bfloat16)
        ones = jnp.ones((1, c.shape[1]), jnp.float32)
        csqh_ref[...] = 0.5 * jax.lax.dot_general(
            ones, c * c,
            dimension_numbers=(((1,), (1,)), ((), ())),
            preferred_element_type=jnp.float32,
            precision=jax.lax.Precision.HIGHEST,
        )

    d = jax.lax.dot_general(
        xf_ref[...].astype(jnp.bfloat16), cb_ref[...],
        dimension_numbers=(((1,), (1,)), ((), ())),
        preferred_element_type=jnp.float32,
    )
    s = d - csqh_ref[...]
    maxv = jnp.max(s, axis=1, keepdims=True)
    out_ref[...] = jnp.where(s == maxv, 1.0, 0.0).astype(jnp.float32)


def kernel(x, cluster_centers):
    H, W, C = x.shape
    K = cluster_centers.shape[0]
    P = H * W
    xf = x.reshape(P, C)
    grid = P // _BP

    out = pl.pallas_call(
        _vq_onehot_body,
        grid=(grid,),
        in_specs=[
            pl.BlockSpec((_BP, C), lambda i: (i, 0)),
            pl.BlockSpec((K, C), lambda i: (0, 0)),
        ],
        out_specs=pl.BlockSpec((_BP, K), lambda i: (i, 0)),
        out_shape=jax.ShapeDtypeStruct((P, K), jnp.float32),
        scratch_shapes=[
            pltpu.VMEM((K, C), jnp.bfloat16),
            pltpu.VMEM((1, K), jnp.float32),
        ],
    )(xf, cluster_centers)
    return out.reshape(H, W, K)


# speedup vs baseline: 2.4381x; 1.2703x over previous
import jax
import jax.numpy as jnp
from jax.experimental import pallas as pl
from jax.experimental.pallas import tpu as pltpu

_BP = 1024


def _vq_onehot_body(xf_ref, c_ref, out_ref, cb_ref, csqh_ref):
    @pl.when(pl.program_id(0) == 0)
    def _():
        c = c_ref[...]
        cb_ref[...] = c.astype(jnp.bfloat16)
        ones = jnp.ones((1, c.shape[1]), jnp.float32)
        csqh_ref[...] = 0.5 * jax.lax.dot_general(
            ones, c * c,
            dimension_numbers=(((1,), (1,)), ((), ())),
            preferred_element_type=jnp.float32,
            precision=jax.lax.Precision.HIGHEST,
        )

    d = jax.lax.dot_general(
        xf_ref[...].astype(jnp.bfloat16), cb_ref[...],
        dimension_numbers=(((1,), (1,)), ((), ())),
        preferred_element_type=jnp.float32,
    )
    s = d - csqh_ref[...]
    maxv = jnp.max(s, axis=1, keepdims=True)
    out_ref[...] = jnp.where(s == maxv, 1.0, 0.0).astype(jnp.float32)


def kernel(x, cluster_centers):
    H, W, C = x.shape
    K = cluster_centers.shape[0]
    P = H * W
    xf = x.reshape(P, C)
    grid = P // _BP

    out = pl.pallas_call(
        _vq_onehot_body,
        grid=(grid,),
        in_specs=[
            pl.BlockSpec((_BP, C), lambda i: (i, 0)),
            pl.BlockSpec((K, C), lambda i: (0, 0)),
        ],
        out_specs=pl.BlockSpec((_BP, K), lambda i: (i, 0)),
        out_shape=jax.ShapeDtypeStruct((P, K), jnp.float32),
        scratch_shapes=[
            pltpu.VMEM((K, C), jnp.bfloat16),
            pltpu.VMEM((1, K), jnp.float32),
        ],
    )(xf, cluster_centers)
    return out.reshape(H, W, K)


# speedup vs baseline: 2.9194x; 1.1974x over previous
import jax
import jax.numpy as jnp
from jax.experimental import pallas as pl
from jax.experimental.pallas import tpu as pltpu

_BP = 3584


def _vq_onehot_body(xf_ref, c_ref, out_ref, cb_ref, csqh_ref):
    @pl.when(pl.program_id(0) == 0)
    def _():
        c = c_ref[...]
        cb_ref[...] = c.astype(jnp.bfloat16)
        ones = jnp.ones((1, c.shape[1]), jnp.float32)
        csqh_ref[...] = 0.5 * jax.lax.dot_general(
            ones, c * c,
            dimension_numbers=(((1,), (1,)), ((), ())),
            preferred_element_type=jnp.float32,
            precision=jax.lax.Precision.HIGHEST,
        )

    d = jax.lax.dot_general(
        xf_ref[...].astype(jnp.bfloat16), cb_ref[...],
        dimension_numbers=(((1,), (1,)), ((), ())),
        preferred_element_type=jnp.float32,
    )
    s = d - csqh_ref[...]
    maxv = jnp.max(s, axis=1, keepdims=True)
    out_ref[...] = jnp.where(s == maxv, 1.0, 0.0).astype(jnp.float32)


def kernel(x, cluster_centers):
    H, W, C = x.shape
    K = cluster_centers.shape[0]
    P = H * W
    xf = x.reshape(P, C)
    grid = P // _BP

    out = pl.pallas_call(
        _vq_onehot_body,
        grid=(grid,),
        in_specs=[
            pl.BlockSpec((_BP, C), lambda i: (i, 0)),
            pl.BlockSpec((K, C), lambda i: (0, 0)),
        ],
        out_specs=pl.BlockSpec((_BP, K), lambda i: (i, 0)),
        out_shape=jax.ShapeDtypeStruct((P, K), jnp.float32),
        scratch_shapes=[
            pltpu.VMEM((K, C), jnp.bfloat16),
            pltpu.VMEM((1, K), jnp.float32),
        ],
    )(xf, cluster_centers)
    return out.reshape(H, W, K)
